# baseline (speedup 1.0000x reference)
import functools

import jax
import jax.numpy as jnp
from jax import lax
from jax.experimental import pallas as pl
from jax.experimental.pallas import tpu as pltpu
from jax.experimental.pallas import tpu_sc as plsc

NE = 8
KSEL = 2
R = 32
SCALE = 64.0 / 32.0


def _stage1_kernel(x_ref, wr1_ref, wr2_ref, aall_ref, logits_ref, xa_ref):
    xb = x_ref[...].astype(jnp.bfloat16)
    h = jnp.maximum(
        jnp.dot(xb, wr1_ref[...], preferred_element_type=jnp.float32), 0.0)
    logits_ref[...] = jnp.dot(h.astype(jnp.bfloat16), wr2_ref[...],
                              preferred_element_type=jnp.float32)
    xa_ref[...] = jnp.dot(xb, aall_ref[...],
                          preferred_element_type=jnp.float32
                          ).astype(jnp.bfloat16)


def _stage3_kernel(x_ref, xa_ref, gates_ref, emat_ref, ssel_ref, b_ref,
                   out_ref):
    x = x_ref[...]
    ge = jnp.dot(gates_ref[...], emat_ref[...],
                 preferred_element_type=jnp.float32)
    weighted = xa_ref[...].astype(jnp.float32) * ge
    combined = jnp.dot(weighted.astype(jnp.bfloat16), ssel_ref[...],
                       preferred_element_type=jnp.float32)
    out_ref[...] = x + jnp.dot(combined.astype(jnp.bfloat16), b_ref[...],
                               preferred_element_type=jnp.float32)


def _gates_on_sc(logitsT):
    E, T = logitsT.shape
    info = plsc.get_sparse_core_info()
    NC, NS, L = info.num_cores, info.num_subcores, info.num_lanes
    NW = NC * NS
    chunk = T // NW
    mesh = plsc.VectorSubcoreMesh(core_axis_name="c", subcore_axis_name="s")

    @functools.partial(
        pl.kernel, mesh=mesh,
        out_type=jax.ShapeDtypeStruct((E, T), jnp.float32),
        scratch_types=[pltpu.VMEM((E, chunk), jnp.float32),
                       pltpu.VMEM((E, chunk), jnp.float32)],
    )
    def k(l_hbm, g_hbm, l_v, g_v):
        wid = lax.axis_index("s") * NC + lax.axis_index("c")
        base = wid * chunk
        pltpu.sync_copy(l_hbm.at[:, pl.ds(base, chunk)], l_v)

        def body(i, carry):
            sl = pl.ds(i * L, L)
            ls = [l_v[e, sl] for e in range(E)]
            v1 = ls[0]
            for e in range(1, E):
                v1 = jnp.maximum(v1, ls[e])
            ninf = jnp.full((L,), -jnp.inf, jnp.float32)
            v2 = ninf
            for e in range(E):
                v2 = jnp.maximum(v2, jnp.where(ls[e] >= v1, ninf, ls[e]))
            w1 = SCALE / (1.0 + jnp.exp(v2 - v1))
            w2 = SCALE - w1
            zero = jnp.zeros((L,), jnp.float32)
            for e in range(E):
                g_v[e, sl] = jnp.where(
                    ls[e] >= v2, jnp.where(ls[e] >= v1, w1, w2), zero)
            return carry

        lax.fori_loop(0, chunk // L, body, 0)
        pltpu.sync_copy(g_v, g_hbm.at[:, pl.ds(base, chunk)])

    return k(logitsT)


@jax.jit
def kernel(x, W_r1, W_r2, A, B):
    T, D = x.shape
    E, _, r = A.shape

    A_all = A.transpose(1, 0, 2).reshape(D, E * r).astype(jnp.bfloat16)
    B = B.astype(jnp.bfloat16)
    W_r1 = W_r1.astype(jnp.bfloat16)
    W_r2 = W_r2.astype(jnp.bfloat16)

    col = jnp.arange(E * r)
    emat = (col[None, :] // r == jnp.arange(E)[:, None]).astype(jnp.float32)
    ssel = (col[:, None] % r == jnp.arange(r)[None, :]).astype(jnp.bfloat16)

    tile = 1024
    grid = (T // tile,)
    full = lambda a: pl.BlockSpec(a.shape, lambda i: (0,) * a.ndim)

    logits, xa = pl.pallas_call(
        _stage1_kernel,
        grid=grid,
        in_specs=[
            pl.BlockSpec((tile, D), lambda i: (i, 0)),
            full(W_r1), full(W_r2), full(A_all),
        ],
        out_specs=[
            pl.BlockSpec((tile, NE), lambda i: (i, 0)),
            pl.BlockSpec((tile, E * r), lambda i: (i, 0)),
        ],
        out_shape=[
            jax.ShapeDtypeStruct((T, NE), jnp.float32),
            jax.ShapeDtypeStruct((T, E * r), jnp.bfloat16),
        ],
        compiler_params=pltpu.CompilerParams(
            dimension_semantics=("arbitrary",),
        ),
    )(x, W_r1, W_r2, A_all)

    gates = _gates_on_sc(logits.T).T

    out = pl.pallas_call(
        _stage3_kernel,
        grid=grid,
        in_specs=[
            pl.BlockSpec((tile, D), lambda i: (i, 0)),
            pl.BlockSpec((tile, E * r), lambda i: (i, 0)),
            pl.BlockSpec((tile, NE), lambda i: (i, 0)),
            full(emat), full(ssel), full(B),
        ],
        out_specs=pl.BlockSpec((tile, D), lambda i: (i, 0)),
        out_shape=jax.ShapeDtypeStruct((T, D), jnp.float32),
        compiler_params=pltpu.CompilerParams(
            dimension_semantics=("arbitrary",),
        ),
    )(x, xa, gates, emat, ssel, B)
    return out


# speedup vs baseline: 1.7663x; 1.7663x over previous
import functools
import math

import jax
import jax.numpy as jnp
from jax.experimental import pallas as pl
from jax.experimental.pallas import tpu as pltpu

NE = 8
KSEL = 2
R = 32
SCALE = 64.0 / 32.0


def _moe_lora_kernel(x_ref, wr1_ref, wr2_ref, aall_ref, b_ref, emat_ref,
                     ssel_ref, out_ref):
    x = x_ref[...]
    xb = x.astype(jnp.bfloat16)

    h = jnp.maximum(
        jnp.dot(xb, wr1_ref[...], preferred_element_type=jnp.float32), 0.0)
    logits = jnp.dot(h.astype(jnp.bfloat16), wr2_ref[...],
                     preferred_element_type=jnp.float32)

    v1 = jnp.max(logits, axis=-1, keepdims=True)
    v2 = jnp.max(jnp.where(logits >= v1, -jnp.inf, logits),
                 axis=-1, keepdims=True)
    w1 = SCALE / (1.0 + jnp.exp(v2 - v1))
    w2 = SCALE - w1
    gates = jnp.where(logits >= v2,
                      jnp.where(logits >= v1, w1, w2), 0.0)

    xa = jnp.dot(xb, aall_ref[...],
                 preferred_element_type=jnp.float32)

    ge = jnp.dot(gates, emat_ref[...], preferred_element_type=jnp.float32)
    combined = jnp.dot(xa * ge, ssel_ref[...],
                       preferred_element_type=jnp.float32)

    out_ref[...] = x + jnp.dot(combined.astype(jnp.bfloat16), b_ref[...],
                               preferred_element_type=jnp.float32)


@jax.jit
def kernel(x, W_r1, W_r2, A, B):
    T, D = x.shape
    E, _, r = A.shape

    A_all = A.transpose(1, 0, 2).reshape(D, E * r).astype(jnp.bfloat16)
    B = B.astype(jnp.bfloat16)
    W_r1 = W_r1.astype(jnp.bfloat16)
    W_r2 = W_r2.astype(jnp.bfloat16)

    col = jnp.arange(E * r)
    emat = (col[None, :] // r == jnp.arange(E)[:, None]).astype(jnp.float32)
    ssel = (col[:, None] % r == jnp.arange(r)[None, :]).astype(jnp.float32)

    tile = 1024
    while T % tile:
        tile //= 2
    grid = (T // tile,)

    full = lambda a: pl.BlockSpec(a.shape, lambda i: (0,) * a.ndim)
    out = pl.pallas_call(
        _moe_lora_kernel,
        grid=grid,
        in_specs=[
            pl.BlockSpec((tile, D), lambda i: (i, 0)),
            full(W_r1), full(W_r2), full(A_all), full(B), full(emat),
            full(ssel),
        ],
        out_specs=pl.BlockSpec((tile, D), lambda i: (i, 0)),
        out_shape=jax.ShapeDtypeStruct((T, D), jnp.float32),
        compiler_params=pltpu.CompilerParams(
            dimension_semantics=("arbitrary",),
        ),
    )(x, W_r1, W_r2, A_all, B, emat, ssel)
    return out


# speedup vs baseline: 1.7709x; 1.0026x over previous
import jax
import jax.numpy as jnp
from jax.experimental import pallas as pl
from jax.experimental.pallas import tpu as pltpu

NE = 8
KSEL = 2
R = 32
SCALE = 64.0 / 32.0


def _moe_lora_kernel(x_ref, wr1_ref, wr2_ref, aall_ref, b_ref, emat_ref,
                     ssel_ref, out_ref):
    x = x_ref[...]
    xb = x.astype(jnp.bfloat16)

    h = jnp.maximum(
        jnp.dot(xb, wr1_ref[...], preferred_element_type=jnp.float32), 0.0)
    logits = jnp.dot(h.astype(jnp.bfloat16), wr2_ref[...],
                     preferred_element_type=jnp.float32)

    v1 = jnp.max(logits, axis=-1, keepdims=True)
    v2 = jnp.max(jnp.where(logits >= v1, -jnp.inf, logits),
                 axis=-1, keepdims=True)
    w1 = SCALE / (1.0 + jnp.exp(v2 - v1))
    w2 = SCALE - w1
    gates = jnp.where(logits >= v2,
                      jnp.where(logits >= v1, w1, w2), 0.0)

    xa = jnp.dot(xb, aall_ref[...],
                 preferred_element_type=jnp.float32)

    ge = jnp.dot(gates, emat_ref[...], preferred_element_type=jnp.float32)
    combined = jnp.dot(xa * ge, ssel_ref[...],
                       preferred_element_type=jnp.float32)

    out_ref[...] = x + jnp.dot(combined.astype(jnp.bfloat16), b_ref[...],
                               preferred_element_type=jnp.float32)


@jax.jit
def kernel(x, W_r1, W_r2, A, B):
    T, D = x.shape
    E, _, r = A.shape

    A_all = A.transpose(1, 0, 2).reshape(D, E * r).astype(jnp.bfloat16)
    B = B.astype(jnp.bfloat16)
    W_r1 = W_r1.astype(jnp.bfloat16)
    W_r2 = W_r2.astype(jnp.bfloat16)

    col = jnp.arange(E * r)
    emat = (col[None, :] // r == jnp.arange(E)[:, None]).astype(jnp.float32)
    ssel = (col[:, None] % r == jnp.arange(r)[None, :]).astype(jnp.float32)

    tile = 1024
    while T % tile:
        tile //= 2
    grid = (T // tile,)

    full = lambda a: pl.BlockSpec(a.shape, lambda i: (0,) * a.ndim)
    out = pl.pallas_call(
        _moe_lora_kernel,
        grid=grid,
        in_specs=[
            pl.BlockSpec((tile, D), lambda i: (i, 0)),
            full(W_r1), full(W_r2), full(A_all), full(B), full(emat),
            full(ssel),
        ],
        out_specs=pl.BlockSpec((tile, D), lambda i: (i, 0)),
        out_shape=jax.ShapeDtypeStruct((T, D), jnp.float32),
        compiler_params=pltpu.CompilerParams(
            dimension_semantics=("parallel",),
        ),
    )(x, W_r1, W_r2, A_all, B, emat, ssel)
    return out
